# baseline (speedup 1.0000x reference)
import functools

import jax
import jax.numpy as jnp
from jax import lax
from jax.experimental import pallas as pl
from jax.experimental.pallas import tpu as pltpu
from jax.experimental.pallas import tpu_sc as plsc

N_NODES = 10000
N_PAD = 10112
D = 128
E = 320000
EPS = 1e-5

NC, NS = 2, 16
NW = NC * NS
EPT = E // NW
K = 80
NCH = EPT // K
RPT = N_PAD // NS
DEGW = 16
BNC = 80
GK = 64
_GROUPS = [(0, 64), (64, 61)]

_BCHUNKS = [(i * BNC, min(BNC, RPT - i * BNC)) for i in range((RPT + BNC - 1) // BNC)]


def _fill_shared(src_vmem, shared, row0):
    for off, sz in _BCHUNKS:
        pltpu.sync_copy(src_vmem.at[pl.ds(0, sz)], shared.at[pl.ds(row0 + off, sz)])


def _readout_shared(shared, bounce, hbm, c, row0):
    for off, sz in _BCHUNKS:
        pltpu.sync_copy(shared.at[pl.ds(row0 + off, sz)], bounce.at[pl.ds(0, sz)])
        pltpu.sync_copy(bounce.at[pl.ds(0, sz)], hbm.at[c, pl.ds(row0 + off, sz)])


def _deg_body(dst_hbm, z128_hbm, ones_hbm,
              deg_hbm,
              deg_sh, dstv, zbuf, onesv):
    c = lax.axis_index("c")
    s = lax.axis_index("s")
    wid = c * NS + s
    row0 = s * RPT

    pltpu.sync_copy(dst_hbm.at[wid], dstv)
    pltpu.sync_copy(z128_hbm, zbuf)
    pltpu.sync_copy(ones_hbm, onesv)
    _fill_shared(zbuf, deg_sh, row0)
    plsc.subcore_barrier()

    def chunk(j, carry):
        pltpu.sync_copy(onesv, deg_sh.at[dstv.at[j]], add=True)
        return carry

    lax.fori_loop(0, NCH, chunk, 0)
    plsc.subcore_barrier()
    _readout_shared(deg_sh, zbuf, deg_hbm, c, row0)


def _deg_hist_body(dst_hbm, z1_hbm,
                   deg_hbm,
                   degp, dstv):
    c = lax.axis_index("c")
    s = lax.axis_index("s")
    wid = c * NS + s

    pltpu.sync_copy(dst_hbm.at[wid], dstv)
    pltpu.sync_copy(z1_hbm, degp)
    ones16 = jnp.ones((16,), jnp.float32)

    def chunk(j, carry):
        for u in range(K // 16):
            idx = dstv[j, pl.ds(u * 16, 16)]
            plsc.addupdate_scatter(degp, [idx], ones16)
        return carry

    lax.fori_loop(0, NCH, chunk, 0)
    pltpu.sync_copy(degp, deg_hbm.at[wid])


def _agg_body(x_hbm, src_hbm, dst_hbm, z128_hbm,
              acc_hbm,
              acc_sh, srcg, dstg, rows0, rows1, rows2, sem0, sem1, sem2):
    c = lax.axis_index("c")
    s = lax.axis_index("s")
    wid = c * NS + s
    row0 = s * RPT

    pltpu.sync_copy(z128_hbm, rows0)
    _fill_shared(rows0, acc_sh, row0)
    plsc.subcore_barrier()

    bufs = (rows0, rows1, rows2)
    sems = (sem0, sem1, sem2)

    def gather(j, b):
        pltpu.async_copy(x_hbm.at[srcg.at[j]], bufs[b], sems[b])

    def gwait(j, b):
        pltpu.make_async_copy(x_hbm.at[srcg.at[j]], bufs[b], sems[b]).wait()

    def scat(j, b):
        pltpu.sync_copy(bufs[b], acc_sh.at[dstg.at[j]], add=True)

    for goff, cnt in _GROUPS:
        pltpu.sync_copy(src_hbm.at[wid, pl.ds(goff, cnt)], srcg.at[pl.ds(0, cnt)])
        pltpu.sync_copy(dst_hbm.at[wid, pl.ds(goff, cnt)], dstg.at[pl.ds(0, cnt)])

        for b in range(2):
            gather(b, b)
        trips = (cnt - 2) // 3

        def trip(q, carry):
            j = 3 * q
            for b in range(3):
                gwait(j + b, b)
                gather(j + b + 2, (b + 2) % 3)
                scat(j + b, b)
            return carry

        lax.fori_loop(0, trips, trip, 0)

        t = 3 * trips
        for r in range(cnt - t):
            b = r % 3
            if r >= 2:
                gather(t + r, b)
            gwait(t + r, b)
            scat(t + r, b)

    plsc.subcore_barrier()
    _readout_shared(acc_sh, rows0, acc_hbm, c, row0)


@functools.cache
def _sc_mesh():
    return plsc.VectorSubcoreMesh(core_axis_name="c", subcore_axis_name="s",
                                  num_cores=NC, num_subcores=NS)


@functools.cache
def _deg_call():
    return pl.kernel(
        _deg_body,
        out_type=jax.ShapeDtypeStruct((NC, N_PAD, D), jnp.float32),
        mesh=_sc_mesh(),
        scratch_types=[
            pltpu.VMEM_SHARED((N_PAD, D), jnp.float32),
            pltpu.VMEM((NCH, K), jnp.int32),
            pltpu.VMEM((BNC, D), jnp.float32),
            pltpu.VMEM((K, D), jnp.float32),
        ],
    )


@functools.cache
def _deg_hist_call():
    return pl.kernel(
        _deg_hist_body,
        out_type=jax.ShapeDtypeStruct((NW, N_PAD), jnp.float32),
        mesh=_sc_mesh(),
        scratch_types=[
            pltpu.VMEM((N_PAD,), jnp.float32),
            pltpu.VMEM((NCH, K), jnp.int32),
        ],
    )


@functools.cache
def _agg_call():
    return pl.kernel(
        _agg_body,
        out_type=jax.ShapeDtypeStruct((NC, N_PAD, D), jnp.float32),
        mesh=_sc_mesh(),
        scratch_types=[
            pltpu.VMEM_SHARED((N_PAD, D), jnp.float32),
            pltpu.VMEM((GK, K), jnp.int32),
            pltpu.VMEM((GK, K), jnp.int32),
            pltpu.VMEM((K, D), jnp.float32),
            pltpu.VMEM((K, D), jnp.float32),
            pltpu.VMEM((K, D), jnp.float32),
            pltpu.SemaphoreType.DMA,
            pltpu.SemaphoreType.DMA,
            pltpu.SemaphoreType.DMA,
        ],
    )


def _rowsum(v):
    return jnp.sum(v, axis=0, keepdims=True)


def _sage_gn(acc, deg, x, wl, bl, wr, gw, gb, gms, mask):
    agg = acc[0] + acc[1]
    mean = agg / jnp.maximum(deg, 1.0)
    pre = (lax.dot_general(mean, wl, (((1,), (1,)), ((), ())),
                           preferred_element_type=jnp.float32)
           + bl[None, :]
           + lax.dot_general(x, wr, (((1,), (1,)), ((), ())),
                             preferred_element_type=jnp.float32))
    n = float(N_NODES)
    cm = _rowsum(pre * mask) / n
    outc = pre - gms[None, :] * cm
    var = _rowsum(outc * outc * mask) / n
    h = gw[None, :] * outc * lax.rsqrt(var + EPS) + gb[None, :]
    h = jnp.where(h >= 0, h, 0.01 * h)
    return h * mask


def _bn_relu(a, w, b, mask):
    n = float(N_NODES)
    m = _rowsum(a * mask) / n
    d = a - m
    var = _rowsum(d * d * mask) / n
    return jnp.maximum(w[None, :] * d * lax.rsqrt(var + EPS) + b[None, :], 0.0)


def _l1a_body(acc_ref, x_ref, wl_ref, wr_ref, m1_ref, rest_ref):
    acc = acc_ref[0] + acc_ref[1]
    m1_ref[...] = lax.dot_general(acc, wl_ref[...], (((1,), (1,)), ((), ())),
                                  preferred_element_type=jnp.float32)
    rest_ref[...] = lax.dot_general(x_ref[...], wr_ref[...],
                                    (((1,), (1,)), ((), ())),
                                    preferred_element_type=jnp.float32)


def _l1b_body(m1_ref, rest_ref, deg_ref, bl_ref, gw_ref, gb_ref, gms_ref,
              o_ref):
    mask = (lax.broadcasted_iota(jnp.int32, (N_PAD, 1), 0)
            < N_NODES).astype(jnp.float32)
    deg = deg_ref[0, :, 0:1] + deg_ref[1, :, 0:1]
    pre = (m1_ref[...] / jnp.maximum(deg, 1.0) + bl_ref[...][None, :]
           + rest_ref[...])
    n = float(N_NODES)
    cm = _rowsum(pre * mask) / n
    outc = pre - gms_ref[...][None, :] * cm
    var = _rowsum(outc * outc * mask) / n
    h = gw_ref[...][None, :] * outc * lax.rsqrt(var + EPS) + gb_ref[...][None, :]
    h = jnp.where(h >= 0, h, 0.01 * h)
    o_ref[...] = h * mask


def _l2_body(acc_ref, deg_ref, h_ref, wl_ref, bl_ref, wr_ref,
             gw_ref, gb_ref, gms_ref,
             m1w_ref, m1b_ref, b1w_ref, b1b_ref,
             m2w_ref, m2b_ref, b2w_ref, b2b_ref,
             m3w_ref, m3b_ref, o_ref):
    mask = (lax.broadcasted_iota(jnp.int32, (N_PAD, 1), 0)
            < N_NODES).astype(jnp.float32)
    deg = deg_ref[0, :, 0:1] + deg_ref[1, :, 0:1]
    h2 = _sage_gn(acc_ref[...], deg, h_ref[...], wl_ref[...], bl_ref[...],
                  wr_ref[...], gw_ref[...], gb_ref[...], gms_ref[...], mask)
    a1 = lax.dot_general(h2, m1w_ref[...], (((1,), (1,)), ((), ())),
                         preferred_element_type=jnp.float32) + m1b_ref[...][None, :]
    r1 = _bn_relu(a1, b1w_ref[...], b1b_ref[...], mask)
    a2 = lax.dot_general(r1, m2w_ref[...], (((1,), (1,)), ((), ())),
                         preferred_element_type=jnp.float32) + m2b_ref[...][None, :]
    r2 = _bn_relu(a2, b2w_ref[...], b2b_ref[...], mask)
    out = jnp.sum(r2 * m3w_ref[0][None, :], axis=1, keepdims=True) + m3b_ref[0]
    o_ref[...] = out


_l1a_call = pl.pallas_call(
    _l1a_body,
    out_shape=(jax.ShapeDtypeStruct((N_PAD, D), jnp.float32),
               jax.ShapeDtypeStruct((N_PAD, D), jnp.float32)),
)

_l1b_call = pl.pallas_call(
    _l1b_body,
    out_shape=jax.ShapeDtypeStruct((N_PAD, D), jnp.float32),
)

_l2_call = pl.pallas_call(
    _l2_body,
    out_shape=jax.ShapeDtypeStruct((N_PAD, 1), jnp.float32),
)


def kernel(x, edge_index, Wl1, bl1, Wr1, gn1_w, gn1_b, gn1_ms,
           Wl2, bl2, Wr2, gn2_w, gn2_b, gn2_ms,
           M1_W, M1_b, bn1_w, bn1_b, M2_W, M2_b, bn2_w, bn2_b, M3_W, M3_b):
    src = edge_index[0].reshape(NW, NCH, K)
    dst = edge_index[1].reshape(NW, NCH, K)
    z128 = jnp.zeros((BNC, D), jnp.float32)
    ones = jnp.ones((K, D), jnp.float32)

    acc1 = _agg_call()(x, src, dst, z128)

    x_pad = jnp.pad(x, ((0, N_PAD - N_NODES), (0, 0)))
    m1, rest = _l1a_call(acc1, x_pad, Wl1, Wr1)
    degacc = _deg_call()(dst, z128, ones)[:, :, :DEGW]
    h1 = _l1b_call(m1, rest, degacc, bl1, gn1_w, gn1_b, gn1_ms)

    acc2 = _agg_call()(h1, src, dst, z128)

    out = _l2_call(acc2, degacc, h1, Wl2, bl2, Wr2, gn2_w, gn2_b, gn2_ms,
                   M1_W, M1_b, bn1_w, bn1_b, M2_W, M2_b, bn2_w, bn2_b,
                   M3_W, M3_b)
    return out[:N_NODES]


# speedup vs baseline: 1.0088x; 1.0088x over previous
import functools

import jax
import jax.numpy as jnp
from jax import lax
from jax.experimental import pallas as pl
from jax.experimental.pallas import tpu as pltpu
from jax.experimental.pallas import tpu_sc as plsc

N_NODES = 10000
N_PAD = 10112
D = 128
E = 320000
EPS = 1e-5

NC, NS = 2, 16
NW = NC * NS
EPT = E // NW
K = 80
NCH = EPT // K
RPT = N_PAD // NS
DEGW = 16
BNC = 80
GK = 64
_GROUPS = [(0, 64), (64, 61)]

_BCHUNKS = [(i * BNC, min(BNC, RPT - i * BNC)) for i in range((RPT + BNC - 1) // BNC)]


def _fill_shared(src_vmem, shared, row0):
    for off, sz in _BCHUNKS:
        pltpu.sync_copy(src_vmem.at[pl.ds(0, sz)], shared.at[pl.ds(row0 + off, sz)])


def _readout_shared(shared, bounce, hbm, c, row0):
    for off, sz in _BCHUNKS:
        pltpu.sync_copy(shared.at[pl.ds(row0 + off, sz)], bounce.at[pl.ds(0, sz)])
        pltpu.sync_copy(bounce.at[pl.ds(0, sz)], hbm.at[c, pl.ds(row0 + off, sz)])


def _deg_body(dst_hbm, z128_hbm, ones_hbm,
              deg_hbm,
              deg_sh, dstv, zbuf, onesv):
    c = lax.axis_index("c")
    s = lax.axis_index("s")
    wid = c * NS + s
    row0 = s * RPT

    pltpu.sync_copy(dst_hbm.at[wid], dstv)
    pltpu.sync_copy(z128_hbm, zbuf)
    pltpu.sync_copy(ones_hbm, onesv)
    _fill_shared(zbuf, deg_sh, row0)
    plsc.subcore_barrier()

    def chunk(j, carry):
        pltpu.sync_copy(onesv, deg_sh.at[dstv.at[j]], add=True)
        return carry

    lax.fori_loop(0, NCH, chunk, 0)
    plsc.subcore_barrier()
    _readout_shared(deg_sh, zbuf, deg_hbm, c, row0)


def _deg_hist_body(dst_hbm, z1_hbm,
                   deg_hbm,
                   degp, dstv):
    c = lax.axis_index("c")
    s = lax.axis_index("s")
    wid = c * NS + s

    pltpu.sync_copy(dst_hbm.at[wid], dstv)
    pltpu.sync_copy(z1_hbm, degp)
    ones16 = jnp.ones((16,), jnp.float32)

    def chunk(j, carry):
        for u in range(K // 16):
            idx = dstv[j, pl.ds(u * 16, 16)]
            plsc.addupdate_scatter(degp, [idx], ones16)
        return carry

    lax.fori_loop(0, NCH, chunk, 0)
    pltpu.sync_copy(degp, deg_hbm.at[wid])


def _agg_body(x_hbm, src_hbm, dst_hbm, z128_hbm,
              acc_hbm,
              acc_sh, srcg, dstg, rows0, rows1, rows2, sem0, sem1, sem2):
    c = lax.axis_index("c")
    s = lax.axis_index("s")
    wid = c * NS + s
    row0 = s * RPT

    pltpu.sync_copy(z128_hbm, rows0)
    _fill_shared(rows0, acc_sh, row0)
    plsc.subcore_barrier()

    bufs = (rows0, rows1, rows2)
    sems = (sem0, sem1, sem2)

    def gather(j, b):
        pltpu.async_copy(x_hbm.at[srcg.at[j]], bufs[b], sems[b])

    def gwait(j, b):
        pltpu.make_async_copy(x_hbm.at[srcg.at[j]], bufs[b], sems[b]).wait()

    def scat(j, b):
        pltpu.sync_copy(bufs[b], acc_sh.at[dstg.at[j]], add=True)

    for goff, cnt in _GROUPS:
        pltpu.sync_copy(src_hbm.at[wid, pl.ds(goff, cnt)], srcg.at[pl.ds(0, cnt)])
        pltpu.sync_copy(dst_hbm.at[wid, pl.ds(goff, cnt)], dstg.at[pl.ds(0, cnt)])

        for b in range(2):
            gather(b, b)
        trips = (cnt - 2) // 3

        def trip(q, carry):
            j = 3 * q
            for b in range(3):
                gwait(j + b, b)
                gather(j + b + 2, (b + 2) % 3)
                scat(j + b, b)
            return carry

        lax.fori_loop(0, trips, trip, 0)

        t = 3 * trips
        for r in range(cnt - t):
            b = r % 3
            if r >= 2:
                gather(t + r, b)
            gwait(t + r, b)
            scat(t + r, b)

    plsc.subcore_barrier()
    _readout_shared(acc_sh, rows0, acc_hbm, c, row0)


@functools.cache
def _sc_mesh():
    return plsc.VectorSubcoreMesh(core_axis_name="c", subcore_axis_name="s",
                                  num_cores=NC, num_subcores=NS)


@functools.cache
def _deg_call():
    return pl.kernel(
        _deg_body,
        out_type=jax.ShapeDtypeStruct((NC, N_PAD, D), jnp.float32),
        mesh=_sc_mesh(),
        scratch_types=[
            pltpu.VMEM_SHARED((N_PAD, D), jnp.float32),
            pltpu.VMEM((NCH, K), jnp.int32),
            pltpu.VMEM((BNC, D), jnp.float32),
            pltpu.VMEM((K, D), jnp.float32),
        ],
    )


@functools.cache
def _deg_hist_call():
    return pl.kernel(
        _deg_hist_body,
        out_type=jax.ShapeDtypeStruct((NW, N_PAD), jnp.float32),
        mesh=_sc_mesh(),
        scratch_types=[
            pltpu.VMEM((N_PAD,), jnp.float32),
            pltpu.VMEM((NCH, K), jnp.int32),
        ],
    )


@functools.cache
def _agg_call():
    return pl.kernel(
        _agg_body,
        out_type=jax.ShapeDtypeStruct((NC, N_PAD, D), jnp.float32),
        mesh=_sc_mesh(),
        scratch_types=[
            pltpu.VMEM_SHARED((N_PAD, D), jnp.float32),
            pltpu.VMEM((GK, K), jnp.int32),
            pltpu.VMEM((GK, K), jnp.int32),
            pltpu.VMEM((K, D), jnp.float32),
            pltpu.VMEM((K, D), jnp.float32),
            pltpu.VMEM((K, D), jnp.float32),
            pltpu.SemaphoreType.DMA,
            pltpu.SemaphoreType.DMA,
            pltpu.SemaphoreType.DMA,
        ],
    )


def _rowsum(v):
    return jnp.sum(v, axis=0, keepdims=True)


def _sage_gn(acc, deg, x, wl, bl, wr, gw, gb, gms, mask):
    agg = acc[0] + acc[1]
    mean = agg / jnp.maximum(deg, 1.0)
    pre = (lax.dot_general(mean, wl, (((1,), (1,)), ((), ())),
                           preferred_element_type=jnp.float32)
           + bl[None, :]
           + lax.dot_general(x, wr, (((1,), (1,)), ((), ())),
                             preferred_element_type=jnp.float32))
    n = float(N_NODES)
    cm = _rowsum(pre * mask) / n
    outc = pre - gms[None, :] * cm
    var = _rowsum(outc * outc * mask) / n
    h = gw[None, :] * outc * lax.rsqrt(var + EPS) + gb[None, :]
    h = jnp.where(h >= 0, h, 0.01 * h)
    return h * mask


def _bn_relu(a, w, b, mask):
    n = float(N_NODES)
    m = _rowsum(a * mask) / n
    d = a - m
    var = _rowsum(d * d * mask) / n
    return jnp.maximum(w[None, :] * d * lax.rsqrt(var + EPS) + b[None, :], 0.0)


def _l1_body(acc_ref, deg_ref, x_ref, wl_ref, bl_ref, wr_ref,
             gw_ref, gb_ref, gms_ref, o_ref):
    mask = (lax.broadcasted_iota(jnp.int32, (N_PAD, 1), 0)
            < N_NODES).astype(jnp.float32)
    deg = deg_ref[0, :, 0:1] + deg_ref[1, :, 0:1]
    o_ref[...] = _sage_gn(acc_ref[...], deg, x_ref[...], wl_ref[...],
                          bl_ref[...], wr_ref[...], gw_ref[...], gb_ref[...],
                          gms_ref[...], mask)


def _l2_body(acc_ref, deg_ref, h_ref, wl_ref, bl_ref, wr_ref,
             gw_ref, gb_ref, gms_ref,
             m1w_ref, m1b_ref, b1w_ref, b1b_ref,
             m2w_ref, m2b_ref, b2w_ref, b2b_ref,
             m3w_ref, m3b_ref, o_ref):
    mask = (lax.broadcasted_iota(jnp.int32, (N_PAD, 1), 0)
            < N_NODES).astype(jnp.float32)
    deg = deg_ref[0, :, 0:1] + deg_ref[1, :, 0:1]
    h2 = _sage_gn(acc_ref[...], deg, h_ref[...], wl_ref[...], bl_ref[...],
                  wr_ref[...], gw_ref[...], gb_ref[...], gms_ref[...], mask)
    a1 = lax.dot_general(h2, m1w_ref[...], (((1,), (1,)), ((), ())),
                         preferred_element_type=jnp.float32) + m1b_ref[...][None, :]
    r1 = _bn_relu(a1, b1w_ref[...], b1b_ref[...], mask)
    a2 = lax.dot_general(r1, m2w_ref[...], (((1,), (1,)), ((), ())),
                         preferred_element_type=jnp.float32) + m2b_ref[...][None, :]
    r2 = _bn_relu(a2, b2w_ref[...], b2b_ref[...], mask)
    out = jnp.sum(r2 * m3w_ref[0][None, :], axis=1, keepdims=True) + m3b_ref[0]
    o_ref[...] = out


_l1_call = pl.pallas_call(
    _l1_body,
    out_shape=jax.ShapeDtypeStruct((N_PAD, D), jnp.float32),
)

_l2_call = pl.pallas_call(
    _l2_body,
    out_shape=jax.ShapeDtypeStruct((N_PAD, 1), jnp.float32),
)


def kernel(x, edge_index, Wl1, bl1, Wr1, gn1_w, gn1_b, gn1_ms,
           Wl2, bl2, Wr2, gn2_w, gn2_b, gn2_ms,
           M1_W, M1_b, bn1_w, bn1_b, M2_W, M2_b, bn2_w, bn2_b, M3_W, M3_b):
    src = edge_index[0].reshape(NW, NCH, K)
    dst = edge_index[1].reshape(NW, NCH, K)
    z128 = jnp.zeros((BNC, D), jnp.float32)
    ones = jnp.ones((K, D), jnp.float32)

    degacc = _deg_call()(dst, z128, ones)[:, :, :DEGW]
    acc1 = _agg_call()(x, src, dst, z128)

    x_pad = jnp.pad(x, ((0, N_PAD - N_NODES), (0, 0)))
    h1 = _l1_call(acc1, degacc, x_pad, Wl1, bl1, Wr1, gn1_w, gn1_b, gn1_ms)

    acc2 = _agg_call()(h1, src, dst, z128)

    out = _l2_call(acc2, degacc, h1, Wl2, bl2, Wr2, gn2_w, gn2_b, gn2_ms,
                   M1_W, M1_b, bn1_w, bn1_b, M2_W, M2_b, bn2_w, bn2_b,
                   M3_W, M3_b)
    return out[:N_NODES]
